# baseline (speedup 1.0000x reference)
import functools

import jax
import jax.numpy as jnp
from jax import lax
from jax.experimental import pallas as pl
from jax.experimental.pallas import tpu as pltpu
from jax.experimental.pallas import tpu_sc as plsc

_BLK = 128


def _sc_gather_rows(table, idx):
    B = idx.shape[0]
    H = table.shape[1]
    info = plsc.get_sparse_core_info()
    nw = info.num_cores * info.num_subcores
    per_w = B // nw
    n_chunks = per_w // _BLK
    mesh = plsc.VectorSubcoreMesh(core_axis_name="c", subcore_axis_name="s")

    @functools.partial(
        pl.kernel,
        mesh=mesh,
        out_type=jax.ShapeDtypeStruct((B, H), jnp.float32),
        scratch_types=[
            pltpu.VMEM((per_w,), jnp.int32),
            pltpu.VMEM((_BLK, H), jnp.float32),
            pltpu.SemaphoreType.DMA,
        ],
    )
    def k(table_hbm, idx_hbm, out_hbm, idx_v, rows_v, sem):
        wid = lax.axis_index("s") * info.num_cores + lax.axis_index("c")
        base = wid * per_w
        pltpu.sync_copy(idx_hbm.at[pl.ds(base, per_w)], idx_v)
        for c in range(n_chunks):
            pltpu.async_copy(
                table_hbm.at[idx_v.at[pl.ds(c * _BLK, _BLK)]], rows_v, sem
            ).wait()
            pltpu.sync_copy(rows_v, out_hbm.at[pl.ds(base + c * _BLK, _BLK)])

    return k(table, idx)


def _sc_gather_rows2(table, idx_a, idx_b):
    B = idx_a.shape[0]
    H = table.shape[1]
    info = plsc.get_sparse_core_info()
    nw = info.num_cores * info.num_subcores
    per_w = B // nw
    n_chunks = per_w // _BLK
    mesh = plsc.VectorSubcoreMesh(core_axis_name="c", subcore_axis_name="s")
    sds = jax.ShapeDtypeStruct((B, H), jnp.float32)

    @functools.partial(
        pl.kernel,
        mesh=mesh,
        out_type=[sds, sds],
        scratch_types=[
            pltpu.VMEM((per_w,), jnp.int32),
            pltpu.VMEM((per_w,), jnp.int32),
            pltpu.VMEM((_BLK, H), jnp.float32),
            pltpu.VMEM((_BLK, H), jnp.float32),
            pltpu.SemaphoreType.DMA,
            pltpu.SemaphoreType.DMA,
        ],
    )
    def k(table_hbm, ia_hbm, ib_hbm, oa_hbm, ob_hbm,
          ia_v, ib_v, ra_v, rb_v, sem_a, sem_b):
        wid = lax.axis_index("s") * info.num_cores + lax.axis_index("c")
        base = wid * per_w
        pltpu.sync_copy(ia_hbm.at[pl.ds(base, per_w)], ia_v)
        pltpu.sync_copy(ib_hbm.at[pl.ds(base, per_w)], ib_v)
        for c in range(n_chunks):
            cp_a = pltpu.async_copy(
                table_hbm.at[ia_v.at[pl.ds(c * _BLK, _BLK)]], ra_v, sem_a)
            cp_b = pltpu.async_copy(
                table_hbm.at[ib_v.at[pl.ds(c * _BLK, _BLK)]], rb_v, sem_b)
            cp_a.wait()
            pltpu.sync_copy(ra_v, oa_hbm.at[pl.ds(base + c * _BLK, _BLK)])
            cp_b.wait()
            pltpu.sync_copy(rb_v, ob_hbm.at[pl.ds(base + c * _BLK, _BLK)])

    return k(table, idx_a, idx_b)


def _bd_body(src_ref, dst_ref, out_ref):
    epb = src_ref.shape[2]
    src = src_ref[0]
    dst = dst_ref[0]
    off = (lax.broadcasted_iota(jnp.int32, (1, epb), 1) // 32) * 16
    sf = src + off
    df = dst + off
    rows = lax.broadcasted_iota(jnp.int32, (_BLK, epb), 0)
    oh_s = (rows == jnp.broadcast_to(sf, (_BLK, epb))).astype(jnp.float32)
    oh_d = (rows == jnp.broadcast_to(df, (_BLK, epb))).astype(jnp.float32)
    out_ref[0] = lax.dot_general(
        oh_d, oh_s, (((1,), (1,)), ((), ())),
        preferred_element_type=jnp.float32)


def _build_bd(edge_index_t, nb, epb):
    src_r = edge_index_t[0].reshape(nb, 1, epb)
    dst_r = edge_index_t[1].reshape(nb, 1, epb)
    return pl.pallas_call(
        _bd_body,
        grid=(nb,),
        in_specs=[
            pl.BlockSpec((1, 1, epb), lambda g: (g, 0, 0)),
            pl.BlockSpec((1, 1, epb), lambda g: (g, 0, 0)),
        ],
        out_specs=pl.BlockSpec((1, _BLK, _BLK), lambda g: (g, 0, 0)),
        out_shape=jax.ShapeDtypeStruct((nb, _BLK, _BLK), jnp.float32),
    )(src_r, dst_r)


def _init_body(gx_ref, lp_ref, rm_ref, wp_ref, bp_ref, lw_ref, lb_ref,
               re_ref, a1_ref, a2_ref, a3_ref, ib_ref, out_ref):
    hn = jnp.dot(gx_ref[...], wp_ref[...],
                 preferred_element_type=jnp.float32) + bp_ref[...]
    hl = lp_ref[...] * lw_ref[...] + lb_ref[...]
    re = re_ref[...]
    hr = re[0:1] + rm_ref[...] * (re[1:2] - re[0:1])
    h = jnp.dot(hn, a1_ref[...], preferred_element_type=jnp.float32)
    h = h + jnp.dot(hl, a2_ref[...], preferred_element_type=jnp.float32)
    h = h + jnp.dot(hr, a3_ref[...], preferred_element_type=jnp.float32)
    out_ref[...] = h + ib_ref[...]


def _initial_h(gx, lpvec, rmask, wp, bp, lw, lb, re, a1, a2, a3, ib, nb):
    full = lambda r, c: pl.BlockSpec((r, c), lambda g: (0, 0))
    return pl.pallas_call(
        _init_body,
        grid=(nb,),
        in_specs=[
            pl.BlockSpec((_BLK, 128), lambda g: (g, 0)),
            pl.BlockSpec((_BLK, 1), lambda g: (g, 0)),
            pl.BlockSpec((_BLK, 1), lambda g: (g, 0)),
            full(128, 128), full(1, 128), full(1, 128), full(1, 128),
            full(2, 128), full(128, 128), full(128, 128), full(128, 128),
            full(1, 128),
        ],
        out_specs=pl.BlockSpec((_BLK, 128), lambda g: (g, 0)),
        out_shape=jax.ShapeDtypeStruct((gx.shape[0], 128), jnp.float32),
    )(gx, lpvec, rmask, wp, bp, lw, lb, re, a1, a2, a3, ib)


def _gin_body(h_ref, bd_ref, e_ref, w1_ref, b1_ref, w2_ref, b2_ref,
              out_ref, cs_ref):
    g = pl.program_id(0)
    hb = h_ref[...]
    agg = jnp.dot(bd_ref[0], hb, preferred_element_type=jnp.float32,
                  precision=lax.Precision.HIGHEST)
    t = hb * e_ref[...] + agg
    z = jnp.dot(t, w1_ref[...], preferred_element_type=jnp.float32)
    z = jnp.maximum(z + b1_ref[...], 0.0)
    out = jnp.dot(z, w2_ref[...], preferred_element_type=jnp.float32)
    out = out + b2_ref[...]
    out_ref[...] = out

    @pl.when(g == 0)
    def _():
        cs_ref[...] = jnp.zeros_like(cs_ref)

    cs_ref[...] += jnp.sum(out, axis=0, keepdims=True)


def _gin(h, bd, evec, w1, b1, w2, b2, nb):
    full = lambda r, c: pl.BlockSpec((r, c), lambda g: (0, 0))
    return pl.pallas_call(
        _gin_body,
        grid=(nb,),
        in_specs=[
            pl.BlockSpec((_BLK, 128), lambda g: (g, 0)),
            pl.BlockSpec((1, _BLK, _BLK), lambda g: (g, 0, 0)),
            full(1, 128), full(128, 128), full(1, 128), full(128, 128),
            full(1, 128),
        ],
        out_specs=[
            pl.BlockSpec((_BLK, 128), lambda g: (g, 0)),
            pl.BlockSpec((1, 128), lambda g: (0, 0)),
        ],
        out_shape=[
            jax.ShapeDtypeStruct((h.shape[0], 128), jnp.float32),
            jax.ShapeDtypeStruct((1, 128), jnp.float32),
        ],
    )(h, bd, evec, w1, b1, w2, b2)


def _cumsum_body(hs_ref, m_ref, t_ref, carry_ref, comp_ref):
    g = pl.program_id(0)

    @pl.when(g == 0)
    def _():
        carry_ref[...] = jnp.zeros_like(carry_ref)
        comp_ref[...] = jnp.zeros_like(comp_ref)

    hb = hs_ref[...] - m_ref[...]
    tri = (lax.broadcasted_iota(jnp.int32, (_BLK, _BLK), 0)
           > lax.broadcasted_iota(jnp.int32, (_BLK, _BLK), 1)
           ).astype(jnp.float32)
    t_ref[...] = carry_ref[...] + jnp.dot(
        tri, hb, preferred_element_type=jnp.float32,
        precision=lax.Precision.HIGHEST)
    carry = carry_ref[...]
    y = jnp.sum(hb, axis=0, keepdims=True) - comp_ref[...]
    t = carry + y
    comp_ref[...] = (t - carry) - y
    carry_ref[...] = t


def _prefix_table(hs, mvec, nb):
    return pl.pallas_call(
        _cumsum_body,
        grid=(nb + 1,),
        in_specs=[
            pl.BlockSpec((_BLK, 128), lambda g: (jnp.minimum(g, nb - 1), 0)),
            pl.BlockSpec((1, 128), lambda g: (0, 0)),
        ],
        out_specs=pl.BlockSpec((_BLK, 128), lambda g: (g, 0)),
        out_shape=jax.ShapeDtypeStruct(((nb + 1) * _BLK, 128), jnp.float32),
        scratch_shapes=[pltpu.VMEM((1, 128), jnp.float32),
                        pltpu.VMEM((1, 128), jnp.float32)],
    )(hs, mvec)


def _mp2_body(h2_ref, te_ref, ts_ref, d_ref, m_ref, wt_ref, wb_ref, b_ref,
              h3_ref, st_ref):
    g = pl.program_id(0)
    h2 = h2_ref[...]
    xc = (te_ref[...] - ts_ref[...]) * d_ref[...] + m_ref[...]
    u = jnp.dot(h2, wt_ref[...], preferred_element_type=jnp.float32)
    u = u + jnp.dot(xc, wb_ref[...], preferred_element_type=jnp.float32)
    h3 = jnp.maximum(u + b_ref[...], 0.0)
    h3_ref[...] = h3

    @pl.when(g == 0)
    def _():
        st_ref[...] = jnp.zeros_like(st_ref)

    st_ref[0:1, :] += jnp.sum(h3, axis=0, keepdims=True)
    st_ref[1:2, :] += jnp.sum(h3 * h3, axis=0, keepdims=True)


def _mp2(h2, te, ts, dcol, mvec, w_top, w_bot, b, nb):
    full = lambda r, c: pl.BlockSpec((r, c), lambda g: (0, 0))
    return pl.pallas_call(
        _mp2_body,
        grid=(nb,),
        in_specs=[
            pl.BlockSpec((_BLK, 128), lambda g: (g, 0)),
            pl.BlockSpec((_BLK, 128), lambda g: (g, 0)),
            pl.BlockSpec((_BLK, 128), lambda g: (g, 0)),
            pl.BlockSpec((_BLK, 1), lambda g: (g, 0)),
            full(1, 128),
            full(128, 128), full(128, 128), full(1, 128),
        ],
        out_specs=[
            pl.BlockSpec((_BLK, 128), lambda g: (g, 0)),
            pl.BlockSpec((2, 128), lambda g: (0, 0)),
        ],
        out_shape=[
            jax.ShapeDtypeStruct((h2.shape[0], 128), jnp.float32),
            jax.ShapeDtypeStruct((2, 128), jnp.float32),
        ],
    )(h2, te, ts, dcol, mvec, w_top, w_bot, b)


def _bn_body(h3_ref, hr_ref, st_ref, g_ref, b_ref, n_ref, out_ref):
    n = n_ref[...]
    mu = st_ref[0:1, :] * (1.0 / n)
    ex2 = st_ref[1:2, :] * (1.0 / n)
    var = ex2 - mu * mu
    inv = lax.rsqrt(var + 1e-5)
    out_ref[...] = ((h3_ref[...] - mu) * inv * g_ref[...] + b_ref[...]
                    + hr_ref[...])


def _bn_residual(h3, hres, stats, gamma, beta, nvec, nb):
    full = lambda r, c: pl.BlockSpec((r, c), lambda g: (0, 0))
    return pl.pallas_call(
        _bn_body,
        grid=(nb,),
        in_specs=[
            pl.BlockSpec((_BLK, 128), lambda g: (g, 0)),
            pl.BlockSpec((_BLK, 128), lambda g: (g, 0)),
            full(2, 128), full(1, 128), full(1, 128), full(1, 128),
        ],
        out_specs=pl.BlockSpec((_BLK, 128), lambda g: (g, 0)),
        out_shape=jax.ShapeDtypeStruct((h3.shape[0], 128), jnp.float32),
    )(h3, hres, stats, gamma, beta, nvec)


def kernel(x, edge_attr, log_probs, node_proj_W, node_proj_b, logp_W, logp_b,
           root_emb, init_W, init_b, gin_eps, gin_W1, gin_b1, gin_W2, gin_b2,
           bn_gamma, bn_beta, mp2_W, mp2_b,
           nodes_t, root_local, edge_ptr, edge_index_t, edge_src_global):
    S, K = nodes_t.shape
    SK = S * K
    H = init_b.shape[0]
    L = gin_eps.shape[0]
    NB = SK // _BLK
    EPB = edge_index_t.shape[1] // NB

    ids = nodes_t.reshape(-1)
    sid, perm = lax.sort(
        (ids, lax.iota(jnp.int32, SK)), dimension=0, num_keys=1)
    s_idx = jnp.searchsorted(sid, ids, side="left").astype(jnp.int32)
    e_idx = jnp.searchsorted(sid, ids, side="right").astype(jnp.int32)
    dcol = (1.0 / (e_idx - s_idx).astype(jnp.float32))[:, None]

    lpvec = jnp.broadcast_to(log_probs[:, None], (S, K)).reshape(SK, 1)
    rmask = (jnp.arange(K, dtype=jnp.int32)[None, :]
             == root_local[:, None]).astype(jnp.float32).reshape(SK, 1)

    A1, A2, A3 = init_W[:H], init_W[H:2 * H], init_W[2 * H:]
    nvec = jnp.full((1, H), float(SK), jnp.float32)

    bd = _build_bd(edge_index_t, NB, EPB)
    gx = _sc_gather_rows(x, ids)
    h = _initial_h(gx, lpvec, rmask, node_proj_W, node_proj_b[None, :],
                   logp_W, logp_b[None, :], root_emb, A1, A2, A3,
                   init_b[None, :], NB)

    for l in range(L):
        hres = h
        evec = jnp.full((1, H), 1.0, jnp.float32) * (1.0 + gin_eps[l])
        h2, cs = _gin(h, bd, evec, gin_W1[l], gin_b1[l][None, :],
                      gin_W2[l], gin_b2[l][None, :], NB)
        mvec = cs * (1.0 / SK)
        hs = _sc_gather_rows(h2, perm)
        t = _prefix_table(hs, mvec, NB)
        te, ts = _sc_gather_rows2(t, e_idx, s_idx)
        h3, stats = _mp2(h2, te, ts, dcol, mvec, mp2_W[l][:H], mp2_W[l][H:],
                         mp2_b[l][None, :], NB)
        h = _bn_residual(h3, hres, stats, bn_gamma[l][None, :],
                         bn_beta[l][None, :], nvec, NB)
    return h


# speedup vs baseline: 1.0079x; 1.0079x over previous
import functools

import jax
import jax.numpy as jnp
from jax import lax
from jax.experimental import pallas as pl
from jax.experimental.pallas import tpu as pltpu
from jax.experimental.pallas import tpu_sc as plsc

_BLK = 128


_NBUF = 4


def _sc_gather_rows(table, idx):
    B = idx.shape[0]
    H = table.shape[1]
    info = plsc.get_sparse_core_info()
    nw = info.num_cores * info.num_subcores
    per_w = B // nw
    n_chunks = per_w // _BLK
    mesh = plsc.VectorSubcoreMesh(core_axis_name="c", subcore_axis_name="s")

    @functools.partial(
        pl.kernel,
        mesh=mesh,
        out_type=jax.ShapeDtypeStruct((B, H), jnp.float32),
        scratch_types=[
            pltpu.VMEM((per_w,), jnp.int32),
            pltpu.VMEM((_NBUF, _BLK, H), jnp.float32),
            pltpu.SemaphoreType.DMA,
            pltpu.SemaphoreType.DMA,
        ],
    )
    def k(table_hbm, idx_hbm, out_hbm, idx_v, rows_v, gsem, wsem):
        wid = lax.axis_index("s") * info.num_cores + lax.axis_index("c")
        base = wid * per_w
        pltpu.sync_copy(idx_hbm.at[pl.ds(base, per_w)], idx_v)
        for g in range(0, n_chunks, _NBUF):
            cps = [
                pltpu.async_copy(
                    table_hbm.at[idx_v.at[pl.ds((g + b) * _BLK, _BLK)]],
                    rows_v.at[b], gsem)
                for b in range(_NBUF)
            ]
            for cp in cps:
                cp.wait()
            wps = [
                pltpu.async_copy(
                    rows_v.at[b],
                    out_hbm.at[pl.ds(base + (g + b) * _BLK, _BLK)], wsem)
                for b in range(_NBUF)
            ]
            for wp in wps:
                wp.wait()

    return k(table, idx)


def _sc_gather_rows2(table, idx_a, idx_b):
    B = idx_a.shape[0]
    H = table.shape[1]
    info = plsc.get_sparse_core_info()
    nw = info.num_cores * info.num_subcores
    per_w = B // nw
    n_chunks = per_w // _BLK
    mesh = plsc.VectorSubcoreMesh(core_axis_name="c", subcore_axis_name="s")
    sds = jax.ShapeDtypeStruct((B, H), jnp.float32)

    @functools.partial(
        pl.kernel,
        mesh=mesh,
        out_type=[sds, sds],
        scratch_types=[
            pltpu.VMEM((per_w,), jnp.int32),
            pltpu.VMEM((per_w,), jnp.int32),
            pltpu.VMEM((2, _BLK, H), jnp.float32),
            pltpu.VMEM((2, _BLK, H), jnp.float32),
            pltpu.SemaphoreType.DMA,
            pltpu.SemaphoreType.DMA,
        ],
    )
    def k(table_hbm, ia_hbm, ib_hbm, oa_hbm, ob_hbm,
          ia_v, ib_v, ra_v, rb_v, gsem, wsem):
        wid = lax.axis_index("s") * info.num_cores + lax.axis_index("c")
        base = wid * per_w
        pltpu.sync_copy(ia_hbm.at[pl.ds(base, per_w)], ia_v)
        pltpu.sync_copy(ib_hbm.at[pl.ds(base, per_w)], ib_v)
        for g in range(0, n_chunks, 2):
            cps = []
            for b in range(2):
                sl = pl.ds((g + b) * _BLK, _BLK)
                cps.append(pltpu.async_copy(
                    table_hbm.at[ia_v.at[sl]], ra_v.at[b], gsem))
                cps.append(pltpu.async_copy(
                    table_hbm.at[ib_v.at[sl]], rb_v.at[b], gsem))
            for cp in cps:
                cp.wait()
            wps = []
            for b in range(2):
                osl = pl.ds(base + (g + b) * _BLK, _BLK)
                wps.append(pltpu.async_copy(ra_v.at[b], oa_hbm.at[osl], wsem))
                wps.append(pltpu.async_copy(rb_v.at[b], ob_hbm.at[osl], wsem))
            for wp in wps:
                wp.wait()

    return k(table, idx_a, idx_b)


def _bd_body(src_ref, dst_ref, out_ref):
    epb = src_ref.shape[2]
    src = src_ref[0]
    dst = dst_ref[0]
    off = (lax.broadcasted_iota(jnp.int32, (1, epb), 1) // 32) * 16
    sf = src + off
    df = dst + off
    rows = lax.broadcasted_iota(jnp.int32, (_BLK, epb), 0)
    oh_s = (rows == jnp.broadcast_to(sf, (_BLK, epb))).astype(jnp.float32)
    oh_d = (rows == jnp.broadcast_to(df, (_BLK, epb))).astype(jnp.float32)
    out_ref[0] = lax.dot_general(
        oh_d, oh_s, (((1,), (1,)), ((), ())),
        preferred_element_type=jnp.float32)


def _build_bd(edge_index_t, nb, epb):
    src_r = edge_index_t[0].reshape(nb, 1, epb)
    dst_r = edge_index_t[1].reshape(nb, 1, epb)
    return pl.pallas_call(
        _bd_body,
        grid=(nb,),
        in_specs=[
            pl.BlockSpec((1, 1, epb), lambda g: (g, 0, 0)),
            pl.BlockSpec((1, 1, epb), lambda g: (g, 0, 0)),
        ],
        out_specs=pl.BlockSpec((1, _BLK, _BLK), lambda g: (g, 0, 0)),
        out_shape=jax.ShapeDtypeStruct((nb, _BLK, _BLK), jnp.float32),
    )(src_r, dst_r)


def _init_body(gx_ref, lp_ref, rm_ref, wp_ref, bp_ref, lw_ref, lb_ref,
               re_ref, a1_ref, a2_ref, a3_ref, ib_ref, out_ref):
    hn = jnp.dot(gx_ref[...], wp_ref[...],
                 preferred_element_type=jnp.float32) + bp_ref[...]
    hl = lp_ref[...] * lw_ref[...] + lb_ref[...]
    re = re_ref[...]
    hr = re[0:1] + rm_ref[...] * (re[1:2] - re[0:1])
    h = jnp.dot(hn, a1_ref[...], preferred_element_type=jnp.float32)
    h = h + jnp.dot(hl, a2_ref[...], preferred_element_type=jnp.float32)
    h = h + jnp.dot(hr, a3_ref[...], preferred_element_type=jnp.float32)
    out_ref[...] = h + ib_ref[...]


def _initial_h(gx, lpvec, rmask, wp, bp, lw, lb, re, a1, a2, a3, ib, nb):
    full = lambda r, c: pl.BlockSpec((r, c), lambda g: (0, 0))
    return pl.pallas_call(
        _init_body,
        grid=(nb,),
        in_specs=[
            pl.BlockSpec((_BLK, 128), lambda g: (g, 0)),
            pl.BlockSpec((_BLK, 1), lambda g: (g, 0)),
            pl.BlockSpec((_BLK, 1), lambda g: (g, 0)),
            full(128, 128), full(1, 128), full(1, 128), full(1, 128),
            full(2, 128), full(128, 128), full(128, 128), full(128, 128),
            full(1, 128),
        ],
        out_specs=pl.BlockSpec((_BLK, 128), lambda g: (g, 0)),
        out_shape=jax.ShapeDtypeStruct((gx.shape[0], 128), jnp.float32),
    )(gx, lpvec, rmask, wp, bp, lw, lb, re, a1, a2, a3, ib)


def _gin_body(h_ref, bd_ref, e_ref, w1_ref, b1_ref, w2_ref, b2_ref,
              out_ref, cs_ref):
    g = pl.program_id(0)
    hb = h_ref[...]
    agg = jnp.dot(bd_ref[0], hb, preferred_element_type=jnp.float32,
                  precision=lax.Precision.HIGHEST)
    t = hb * e_ref[...] + agg
    z = jnp.dot(t, w1_ref[...], preferred_element_type=jnp.float32)
    z = jnp.maximum(z + b1_ref[...], 0.0)
    out = jnp.dot(z, w2_ref[...], preferred_element_type=jnp.float32)
    out = out + b2_ref[...]
    out_ref[...] = out

    @pl.when(g == 0)
    def _():
        cs_ref[...] = jnp.zeros_like(cs_ref)

    cs_ref[...] += jnp.sum(out, axis=0, keepdims=True)


def _gin(h, bd, evec, w1, b1, w2, b2, nb):
    full = lambda r, c: pl.BlockSpec((r, c), lambda g: (0, 0))
    return pl.pallas_call(
        _gin_body,
        grid=(nb,),
        in_specs=[
            pl.BlockSpec((_BLK, 128), lambda g: (g, 0)),
            pl.BlockSpec((1, _BLK, _BLK), lambda g: (g, 0, 0)),
            full(1, 128), full(128, 128), full(1, 128), full(128, 128),
            full(1, 128),
        ],
        out_specs=[
            pl.BlockSpec((_BLK, 128), lambda g: (g, 0)),
            pl.BlockSpec((1, 128), lambda g: (0, 0)),
        ],
        out_shape=[
            jax.ShapeDtypeStruct((h.shape[0], 128), jnp.float32),
            jax.ShapeDtypeStruct((1, 128), jnp.float32),
        ],
    )(h, bd, evec, w1, b1, w2, b2)


def _cumsum_body(hs_ref, m_ref, t_ref, carry_ref, comp_ref):
    g = pl.program_id(0)

    @pl.when(g == 0)
    def _():
        carry_ref[...] = jnp.zeros_like(carry_ref)
        comp_ref[...] = jnp.zeros_like(comp_ref)

    hb = hs_ref[...] - m_ref[...]
    tri = (lax.broadcasted_iota(jnp.int32, (_BLK, _BLK), 0)
           > lax.broadcasted_iota(jnp.int32, (_BLK, _BLK), 1)
           ).astype(jnp.float32)
    t_ref[...] = carry_ref[...] + jnp.dot(
        tri, hb, preferred_element_type=jnp.float32,
        precision=lax.Precision.HIGHEST)
    carry = carry_ref[...]
    y = jnp.sum(hb, axis=0, keepdims=True) - comp_ref[...]
    t = carry + y
    comp_ref[...] = (t - carry) - y
    carry_ref[...] = t


def _prefix_table(hs, mvec, nb):
    return pl.pallas_call(
        _cumsum_body,
        grid=(nb + 1,),
        in_specs=[
            pl.BlockSpec((_BLK, 128), lambda g: (jnp.minimum(g, nb - 1), 0)),
            pl.BlockSpec((1, 128), lambda g: (0, 0)),
        ],
        out_specs=pl.BlockSpec((_BLK, 128), lambda g: (g, 0)),
        out_shape=jax.ShapeDtypeStruct(((nb + 1) * _BLK, 128), jnp.float32),
        scratch_shapes=[pltpu.VMEM((1, 128), jnp.float32),
                        pltpu.VMEM((1, 128), jnp.float32)],
    )(hs, mvec)


def _mp2_body(h2_ref, te_ref, ts_ref, d_ref, m_ref, wt_ref, wb_ref, b_ref,
              h3_ref, st_ref):
    g = pl.program_id(0)
    h2 = h2_ref[...]
    xc = (te_ref[...] - ts_ref[...]) * d_ref[...] + m_ref[...]
    u = jnp.dot(h2, wt_ref[...], preferred_element_type=jnp.float32)
    u = u + jnp.dot(xc, wb_ref[...], preferred_element_type=jnp.float32)
    h3 = jnp.maximum(u + b_ref[...], 0.0)
    h3_ref[...] = h3

    @pl.when(g == 0)
    def _():
        st_ref[...] = jnp.zeros_like(st_ref)

    st_ref[0:1, :] += jnp.sum(h3, axis=0, keepdims=True)
    st_ref[1:2, :] += jnp.sum(h3 * h3, axis=0, keepdims=True)


def _mp2(h2, te, ts, dcol, mvec, w_top, w_bot, b, nb):
    full = lambda r, c: pl.BlockSpec((r, c), lambda g: (0, 0))
    return pl.pallas_call(
        _mp2_body,
        grid=(nb,),
        in_specs=[
            pl.BlockSpec((_BLK, 128), lambda g: (g, 0)),
            pl.BlockSpec((_BLK, 128), lambda g: (g, 0)),
            pl.BlockSpec((_BLK, 128), lambda g: (g, 0)),
            pl.BlockSpec((_BLK, 1), lambda g: (g, 0)),
            full(1, 128),
            full(128, 128), full(128, 128), full(1, 128),
        ],
        out_specs=[
            pl.BlockSpec((_BLK, 128), lambda g: (g, 0)),
            pl.BlockSpec((2, 128), lambda g: (0, 0)),
        ],
        out_shape=[
            jax.ShapeDtypeStruct((h2.shape[0], 128), jnp.float32),
            jax.ShapeDtypeStruct((2, 128), jnp.float32),
        ],
    )(h2, te, ts, dcol, mvec, w_top, w_bot, b)


def _bn_body(h3_ref, hr_ref, st_ref, g_ref, b_ref, n_ref, out_ref):
    n = n_ref[...]
    mu = st_ref[0:1, :] * (1.0 / n)
    ex2 = st_ref[1:2, :] * (1.0 / n)
    var = ex2 - mu * mu
    inv = lax.rsqrt(var + 1e-5)
    out_ref[...] = ((h3_ref[...] - mu) * inv * g_ref[...] + b_ref[...]
                    + hr_ref[...])


def _bn_residual(h3, hres, stats, gamma, beta, nvec, nb):
    full = lambda r, c: pl.BlockSpec((r, c), lambda g: (0, 0))
    return pl.pallas_call(
        _bn_body,
        grid=(nb,),
        in_specs=[
            pl.BlockSpec((_BLK, 128), lambda g: (g, 0)),
            pl.BlockSpec((_BLK, 128), lambda g: (g, 0)),
            full(2, 128), full(1, 128), full(1, 128), full(1, 128),
        ],
        out_specs=pl.BlockSpec((_BLK, 128), lambda g: (g, 0)),
        out_shape=jax.ShapeDtypeStruct((h3.shape[0], 128), jnp.float32),
    )(h3, hres, stats, gamma, beta, nvec)


def kernel(x, edge_attr, log_probs, node_proj_W, node_proj_b, logp_W, logp_b,
           root_emb, init_W, init_b, gin_eps, gin_W1, gin_b1, gin_W2, gin_b2,
           bn_gamma, bn_beta, mp2_W, mp2_b,
           nodes_t, root_local, edge_ptr, edge_index_t, edge_src_global):
    S, K = nodes_t.shape
    SK = S * K
    H = init_b.shape[0]
    L = gin_eps.shape[0]
    NB = SK // _BLK
    EPB = edge_index_t.shape[1] // NB

    ids = nodes_t.reshape(-1)
    sid, perm = lax.sort(
        (ids, lax.iota(jnp.int32, SK)), dimension=0, num_keys=1)
    s_idx = jnp.searchsorted(sid, ids, side="left").astype(jnp.int32)
    e_idx = jnp.searchsorted(sid, ids, side="right").astype(jnp.int32)
    dcol = (1.0 / (e_idx - s_idx).astype(jnp.float32))[:, None]

    lpvec = jnp.broadcast_to(log_probs[:, None], (S, K)).reshape(SK, 1)
    rmask = (jnp.arange(K, dtype=jnp.int32)[None, :]
             == root_local[:, None]).astype(jnp.float32).reshape(SK, 1)

    A1, A2, A3 = init_W[:H], init_W[H:2 * H], init_W[2 * H:]
    nvec = jnp.full((1, H), float(SK), jnp.float32)

    bd = _build_bd(edge_index_t, NB, EPB)
    gx = _sc_gather_rows(x, ids)
    h = _initial_h(gx, lpvec, rmask, node_proj_W, node_proj_b[None, :],
                   logp_W, logp_b[None, :], root_emb, A1, A2, A3,
                   init_b[None, :], NB)

    for l in range(L):
        hres = h
        evec = jnp.full((1, H), 1.0, jnp.float32) * (1.0 + gin_eps[l])
        h2, cs = _gin(h, bd, evec, gin_W1[l], gin_b1[l][None, :],
                      gin_W2[l], gin_b2[l][None, :], NB)
        mvec = cs * (1.0 / SK)
        hs = _sc_gather_rows(h2, perm)
        t = _prefix_table(hs, mvec, NB)
        te, ts = _sc_gather_rows2(t, e_idx, s_idx)
        h3, stats = _mp2(h2, te, ts, dcol, mvec, mp2_W[l][:H], mp2_W[l][H:],
                         mp2_b[l][None, :], NB)
        h = _bn_residual(h3, hres, stats, bn_gamma[l][None, :],
                         bn_beta[l][None, :], nvec, NB)
    return h


# speedup vs baseline: 1.1894x; 1.1801x over previous
import functools

import jax
import jax.numpy as jnp
from jax import lax
from jax.experimental import pallas as pl
from jax.experimental.pallas import tpu as pltpu
from jax.experimental.pallas import tpu_sc as plsc

_BLK = 128


_NBUF = 4


def _sc_gather_rows(table, idx):
    B = idx.shape[0]
    H = table.shape[1]
    info = plsc.get_sparse_core_info()
    nw = info.num_cores * info.num_subcores
    per_w = B // nw
    n_chunks = per_w // _BLK
    mesh = plsc.VectorSubcoreMesh(core_axis_name="c", subcore_axis_name="s")

    @functools.partial(
        pl.kernel,
        mesh=mesh,
        out_type=jax.ShapeDtypeStruct((B, H), jnp.float32),
        scratch_types=[
            pltpu.VMEM((per_w,), jnp.int32),
            pltpu.VMEM((_NBUF, _BLK, H), jnp.float32),
            pltpu.SemaphoreType.DMA,
            pltpu.SemaphoreType.DMA,
        ],
    )
    def k(table_hbm, idx_hbm, out_hbm, idx_v, rows_v, gsem, wsem):
        wid = lax.axis_index("s") * info.num_cores + lax.axis_index("c")
        base = wid * per_w
        pltpu.sync_copy(idx_hbm.at[pl.ds(base, per_w)], idx_v)
        for g in range(0, n_chunks, _NBUF):
            cps = [
                pltpu.async_copy(
                    table_hbm.at[idx_v.at[pl.ds((g + b) * _BLK, _BLK)]],
                    rows_v.at[b], gsem)
                for b in range(_NBUF)
            ]
            for cp in cps:
                cp.wait()
            wps = [
                pltpu.async_copy(
                    rows_v.at[b],
                    out_hbm.at[pl.ds(base + (g + b) * _BLK, _BLK)], wsem)
                for b in range(_NBUF)
            ]
            for wp in wps:
                wp.wait()

    return k(table, idx)


def _sc_gather_rows2(table, idx_a, idx_b):
    B = idx_a.shape[0]
    H = table.shape[1]
    info = plsc.get_sparse_core_info()
    nw = info.num_cores * info.num_subcores
    per_w = B // nw
    n_chunks = per_w // _BLK
    mesh = plsc.VectorSubcoreMesh(core_axis_name="c", subcore_axis_name="s")
    sds = jax.ShapeDtypeStruct((B, H), jnp.float32)

    @functools.partial(
        pl.kernel,
        mesh=mesh,
        out_type=[sds, sds],
        scratch_types=[
            pltpu.VMEM((per_w,), jnp.int32),
            pltpu.VMEM((per_w,), jnp.int32),
            pltpu.VMEM((2, _BLK, H), jnp.float32),
            pltpu.VMEM((2, _BLK, H), jnp.float32),
            pltpu.SemaphoreType.DMA,
            pltpu.SemaphoreType.DMA,
        ],
    )
    def k(table_hbm, ia_hbm, ib_hbm, oa_hbm, ob_hbm,
          ia_v, ib_v, ra_v, rb_v, gsem, wsem):
        wid = lax.axis_index("s") * info.num_cores + lax.axis_index("c")
        base = wid * per_w
        pltpu.sync_copy(ia_hbm.at[pl.ds(base, per_w)], ia_v)
        pltpu.sync_copy(ib_hbm.at[pl.ds(base, per_w)], ib_v)
        for g in range(0, n_chunks, 2):
            cps = []
            for b in range(2):
                sl = pl.ds((g + b) * _BLK, _BLK)
                cps.append(pltpu.async_copy(
                    table_hbm.at[ia_v.at[sl]], ra_v.at[b], gsem))
                cps.append(pltpu.async_copy(
                    table_hbm.at[ib_v.at[sl]], rb_v.at[b], gsem))
            for cp in cps:
                cp.wait()
            wps = []
            for b in range(2):
                osl = pl.ds(base + (g + b) * _BLK, _BLK)
                wps.append(pltpu.async_copy(ra_v.at[b], oa_hbm.at[osl], wsem))
                wps.append(pltpu.async_copy(rb_v.at[b], ob_hbm.at[osl], wsem))
            for wp in wps:
                wp.wait()

    return k(table, idx_a, idx_b)


def _bd_body(src_ref, dst_ref, out_ref):
    epb = src_ref.shape[2]
    src = src_ref[0]
    dst = dst_ref[0]
    off = (lax.broadcasted_iota(jnp.int32, (1, epb), 1) // 32) * 16
    sf = src + off
    df = dst + off
    rows = lax.broadcasted_iota(jnp.int32, (_BLK, epb), 0)
    oh_s = (rows == jnp.broadcast_to(sf, (_BLK, epb))).astype(jnp.float32)
    oh_d = (rows == jnp.broadcast_to(df, (_BLK, epb))).astype(jnp.float32)
    out_ref[0] = lax.dot_general(
        oh_d, oh_s, (((1,), (1,)), ((), ())),
        preferred_element_type=jnp.float32)


def _build_bd(edge_index_t, nb, epb):
    src_r = edge_index_t[0].reshape(nb, 1, epb)
    dst_r = edge_index_t[1].reshape(nb, 1, epb)
    return pl.pallas_call(
        _bd_body,
        grid=(nb,),
        in_specs=[
            pl.BlockSpec((1, 1, epb), lambda g: (g, 0, 0)),
            pl.BlockSpec((1, 1, epb), lambda g: (g, 0, 0)),
        ],
        out_specs=pl.BlockSpec((1, _BLK, _BLK), lambda g: (g, 0, 0)),
        out_shape=jax.ShapeDtypeStruct((nb, _BLK, _BLK), jnp.float32),
    )(src_r, dst_r)


def _init_body(gx_ref, lp_ref, rm_ref, wp_ref, bp_ref, lw_ref, lb_ref,
               re_ref, a1_ref, a2_ref, a3_ref, ib_ref, out_ref):
    hn = jnp.dot(gx_ref[...], wp_ref[...],
                 preferred_element_type=jnp.float32) + bp_ref[...]
    hl = lp_ref[...] * lw_ref[...] + lb_ref[...]
    re = re_ref[...]
    hr = re[0:1] + rm_ref[...] * (re[1:2] - re[0:1])
    h = jnp.dot(hn, a1_ref[...], preferred_element_type=jnp.float32)
    h = h + jnp.dot(hl, a2_ref[...], preferred_element_type=jnp.float32)
    h = h + jnp.dot(hr, a3_ref[...], preferred_element_type=jnp.float32)
    out_ref[...] = h + ib_ref[...]


def _initial_h(gx, lpvec, rmask, wp, bp, lw, lb, re, a1, a2, a3, ib, nb):
    full = lambda r, c: pl.BlockSpec((r, c), lambda g: (0, 0))
    return pl.pallas_call(
        _init_body,
        grid=(nb,),
        in_specs=[
            pl.BlockSpec((_BLK, 128), lambda g: (g, 0)),
            pl.BlockSpec((_BLK, 1), lambda g: (g, 0)),
            pl.BlockSpec((_BLK, 1), lambda g: (g, 0)),
            full(128, 128), full(1, 128), full(1, 128), full(1, 128),
            full(2, 128), full(128, 128), full(128, 128), full(128, 128),
            full(1, 128),
        ],
        out_specs=pl.BlockSpec((_BLK, 128), lambda g: (g, 0)),
        out_shape=jax.ShapeDtypeStruct((gx.shape[0], 128), jnp.float32),
    )(gx, lpvec, rmask, wp, bp, lw, lb, re, a1, a2, a3, ib)


def _gin_body(h_ref, bd_ref, e_ref, w1_ref, b1_ref, w2_ref, b2_ref,
              out_ref, cs_ref):
    g = pl.program_id(0)
    hb = h_ref[...]
    agg = jnp.dot(bd_ref[0], hb, preferred_element_type=jnp.float32,
                  precision=lax.Precision.HIGHEST)
    t = hb * e_ref[...] + agg
    z = jnp.dot(t, w1_ref[...], preferred_element_type=jnp.float32)
    z = jnp.maximum(z + b1_ref[...], 0.0)
    out = jnp.dot(z, w2_ref[...], preferred_element_type=jnp.float32)
    out = out + b2_ref[...]
    out_ref[...] = out

    @pl.when(g == 0)
    def _():
        cs_ref[...] = jnp.zeros_like(cs_ref)

    cs_ref[...] += jnp.sum(out, axis=0, keepdims=True)


def _gin(h, bd, evec, w1, b1, w2, b2, nb):
    full = lambda r, c: pl.BlockSpec((r, c), lambda g: (0, 0))
    return pl.pallas_call(
        _gin_body,
        grid=(nb,),
        in_specs=[
            pl.BlockSpec((_BLK, 128), lambda g: (g, 0)),
            pl.BlockSpec((1, _BLK, _BLK), lambda g: (g, 0, 0)),
            full(1, 128), full(128, 128), full(1, 128), full(128, 128),
            full(1, 128),
        ],
        out_specs=[
            pl.BlockSpec((_BLK, 128), lambda g: (g, 0)),
            pl.BlockSpec((1, 128), lambda g: (0, 0)),
        ],
        out_shape=[
            jax.ShapeDtypeStruct((h.shape[0], 128), jnp.float32),
            jax.ShapeDtypeStruct((1, 128), jnp.float32),
        ],
    )(h, bd, evec, w1, b1, w2, b2)


def _cumsum_body(hs_ref, m_ref, t_ref, carry_ref, comp_ref):
    g = pl.program_id(0)

    @pl.when(g == 0)
    def _():
        carry_ref[...] = jnp.zeros_like(carry_ref)
        comp_ref[...] = jnp.zeros_like(comp_ref)

    hb = hs_ref[...] - m_ref[...]
    tri = (lax.broadcasted_iota(jnp.int32, (_BLK, _BLK), 0)
           > lax.broadcasted_iota(jnp.int32, (_BLK, _BLK), 1)
           ).astype(jnp.float32)
    t_ref[...] = carry_ref[...] + jnp.dot(
        tri, hb, preferred_element_type=jnp.float32,
        precision=lax.Precision.HIGHEST)
    carry = carry_ref[...]
    y = jnp.sum(hb, axis=0, keepdims=True) - comp_ref[...]
    t = carry + y
    comp_ref[...] = (t - carry) - y
    carry_ref[...] = t


def _prefix_table(hs, mvec, nb):
    return pl.pallas_call(
        _cumsum_body,
        grid=(nb + 1,),
        in_specs=[
            pl.BlockSpec((_BLK, 128), lambda g: (jnp.minimum(g, nb - 1), 0)),
            pl.BlockSpec((1, 128), lambda g: (0, 0)),
        ],
        out_specs=pl.BlockSpec((_BLK, 128), lambda g: (g, 0)),
        out_shape=jax.ShapeDtypeStruct(((nb + 1) * _BLK, 128), jnp.float32),
        scratch_shapes=[pltpu.VMEM((1, 128), jnp.float32),
                        pltpu.VMEM((1, 128), jnp.float32)],
    )(hs, mvec)


def _mp2_body(h2_ref, te_ref, ts_ref, d_ref, m_ref, wt_ref, wb_ref, b_ref,
              h3_ref, st_ref):
    g = pl.program_id(0)
    h2 = h2_ref[...]
    xc = (te_ref[...] - ts_ref[...]) * d_ref[...] + m_ref[...]
    u = jnp.dot(h2, wt_ref[...], preferred_element_type=jnp.float32)
    u = u + jnp.dot(xc, wb_ref[...], preferred_element_type=jnp.float32)
    h3 = jnp.maximum(u + b_ref[...], 0.0)
    h3_ref[...] = h3

    @pl.when(g == 0)
    def _():
        st_ref[...] = jnp.zeros_like(st_ref)

    st_ref[0:1, :] += jnp.sum(h3, axis=0, keepdims=True)
    st_ref[1:2, :] += jnp.sum(h3 * h3, axis=0, keepdims=True)


def _mp2(h2, te, ts, dcol, mvec, w_top, w_bot, b, nb):
    full = lambda r, c: pl.BlockSpec((r, c), lambda g: (0, 0))
    return pl.pallas_call(
        _mp2_body,
        grid=(nb,),
        in_specs=[
            pl.BlockSpec((_BLK, 128), lambda g: (g, 0)),
            pl.BlockSpec((_BLK, 128), lambda g: (g, 0)),
            pl.BlockSpec((_BLK, 128), lambda g: (g, 0)),
            pl.BlockSpec((_BLK, 1), lambda g: (g, 0)),
            full(1, 128),
            full(128, 128), full(128, 128), full(1, 128),
        ],
        out_specs=[
            pl.BlockSpec((_BLK, 128), lambda g: (g, 0)),
            pl.BlockSpec((2, 128), lambda g: (0, 0)),
        ],
        out_shape=[
            jax.ShapeDtypeStruct((h2.shape[0], 128), jnp.float32),
            jax.ShapeDtypeStruct((2, 128), jnp.float32),
        ],
    )(h2, te, ts, dcol, mvec, w_top, w_bot, b)


def _bn_body(h3_ref, hr_ref, st_ref, g_ref, b_ref, n_ref, out_ref):
    n = n_ref[...]
    mu = st_ref[0:1, :] * (1.0 / n)
    ex2 = st_ref[1:2, :] * (1.0 / n)
    var = ex2 - mu * mu
    inv = lax.rsqrt(var + 1e-5)
    out_ref[...] = ((h3_ref[...] - mu) * inv * g_ref[...] + b_ref[...]
                    + hr_ref[...])


def _bn_residual(h3, hres, stats, gamma, beta, nvec, nb):
    full = lambda r, c: pl.BlockSpec((r, c), lambda g: (0, 0))
    return pl.pallas_call(
        _bn_body,
        grid=(nb,),
        in_specs=[
            pl.BlockSpec((_BLK, 128), lambda g: (g, 0)),
            pl.BlockSpec((_BLK, 128), lambda g: (g, 0)),
            full(2, 128), full(1, 128), full(1, 128), full(1, 128),
        ],
        out_specs=pl.BlockSpec((_BLK, 128), lambda g: (g, 0)),
        out_shape=jax.ShapeDtypeStruct((h3.shape[0], 128), jnp.float32),
    )(h3, hres, stats, gamma, beta, nvec)


def kernel(x, edge_attr, log_probs, node_proj_W, node_proj_b, logp_W, logp_b,
           root_emb, init_W, init_b, gin_eps, gin_W1, gin_b1, gin_W2, gin_b2,
           bn_gamma, bn_beta, mp2_W, mp2_b,
           nodes_t, root_local, edge_ptr, edge_index_t, edge_src_global):
    S, K = nodes_t.shape
    SK = S * K
    H = init_b.shape[0]
    L = gin_eps.shape[0]
    NB = SK // _BLK
    EPB = edge_index_t.shape[1] // NB

    ids = nodes_t.reshape(-1)
    sid, perm = lax.sort(
        (ids, lax.iota(jnp.int32, SK)), dimension=0, num_keys=1)
    j = lax.iota(jnp.int32, SK)
    is_start = jnp.concatenate(
        [jnp.ones((1,), jnp.bool_), sid[1:] != sid[:-1]])
    s_s = lax.cummax(jnp.where(is_start, j, 0), axis=0)
    nxt = jnp.where(is_start, j, SK)
    brr = jnp.concatenate([nxt[1:], jnp.full((1,), SK, jnp.int32)])
    e_s = lax.cummin(brr, axis=0, reverse=True)
    se = jnp.stack([s_s, e_s], axis=1)
    se_o = jnp.zeros((SK, 2), jnp.int32).at[perm].set(se)
    s_idx, e_idx = se_o[:, 0], se_o[:, 1]
    dcol = (1.0 / (e_idx - s_idx).astype(jnp.float32))[:, None]

    lpvec = jnp.broadcast_to(log_probs[:, None], (S, K)).reshape(SK, 1)
    rmask = (jnp.arange(K, dtype=jnp.int32)[None, :]
             == root_local[:, None]).astype(jnp.float32).reshape(SK, 1)

    A1, A2, A3 = init_W[:H], init_W[H:2 * H], init_W[2 * H:]
    nvec = jnp.full((1, H), float(SK), jnp.float32)

    bd = _build_bd(edge_index_t, NB, EPB)
    gx = _sc_gather_rows(x, ids)
    h = _initial_h(gx, lpvec, rmask, node_proj_W, node_proj_b[None, :],
                   logp_W, logp_b[None, :], root_emb, A1, A2, A3,
                   init_b[None, :], NB)

    for l in range(L):
        hres = h
        evec = jnp.full((1, H), 1.0, jnp.float32) * (1.0 + gin_eps[l])
        h2, cs = _gin(h, bd, evec, gin_W1[l], gin_b1[l][None, :],
                      gin_W2[l], gin_b2[l][None, :], NB)
        mvec = cs * (1.0 / SK)
        hs = _sc_gather_rows(h2, perm)
        t = _prefix_table(hs, mvec, NB)
        te, ts = _sc_gather_rows2(t, e_idx, s_idx)
        h3, stats = _mp2(h2, te, ts, dcol, mvec, mp2_W[l][:H], mp2_W[l][H:],
                         mp2_b[l][None, :], NB)
        h = _bn_residual(h3, hres, stats, bn_gamma[l][None, :],
                         bn_beta[l][None, :], nvec, NB)
    return h
